# baseline (speedup 1.0000x reference)
import jax
import jax.numpy as jnp
from jax import lax
from jax.experimental import pallas as pl
from jax.experimental.pallas import tpu as pltpu
from jax.experimental.pallas import tpu_sc as plsc

N_NODES = 10000
N_EDGES = 320000
N_GRAPHS = 16
N_EXPERTS = 8
HID = 128
OUT = 6

_NC = 2
_NS = 16

_HALF = HID // 2
_CHUNK = 128
_E_PER_T = 20480
_E_PAD = _NS * _E_PER_T
_NCH = _E_PER_T // _CHUNK
_NBUF = 6
_NBUF_S = 3
_QCH = 40

_AGG_ROWS = 10240
_ROWS_PER_TILE = _AGG_ROWS // _NS
_DUMP = _AGG_ROWS - 1

_F32 = jnp.float32
_HIGH = jax.lax.Precision.HIGHEST


def _sc_mesh():
    return plsc.VectorSubcoreMesh(core_axis_name="c", subcore_axis_name="s")


def _edge_pass(table, src_all, dst_all, rows, agg, gsems, ssems, nbuf, nch):
    g = [None] * nbuf
    sc = [None] * nbuf

    def start_gather(j):
        b = j % nbuf
        g[b] = pltpu.async_copy(table.at[src_all.at[j]], rows.at[b], gsems[b])

    def start_scatter(j):
        b = j % nbuf
        sc[b] = pltpu.async_copy(rows.at[b], agg.at[dst_all.at[j]], ssems[b],
                                 add=True)

    for j in range(nbuf - 1):
        start_gather(j)
    for j in range(nch):
        b = j % nbuf
        g[b].wait()
        start_scatter(j)
        jn = j + nbuf - 1
        if jn < nch:
            bn = jn % nbuf
            if sc[bn] is not None:
                sc[bn].wait()
            start_gather(jn)
    for j in range(nch - nbuf, nch):
        if sc[j % nbuf] is not None:
            sc[j % nbuf].wait()


def _writeback_and_zero(out_slot, zrows, rows, agg, c, s, zero_after, sem):
    h = pltpu.async_copy(
        agg.at[pl.ds(s * _ROWS_PER_TILE, _ROWS_PER_TILE)],
        out_slot.at[pl.ds(s * _ROWS_PER_TILE, _ROWS_PER_TILE)], sem)
    h.wait()
    if zero_after:
        pltpu.sync_copy(
            zrows, agg.at[pl.ds(s * _ROWS_PER_TILE, _ROWS_PER_TILE)])
        plsc.subcore_barrier()


def _make_multi_body(ntab):
    def body(*refs):
        tables = refs[:ntab]
        src2d, dstix, zrows, out = refs[ntab:ntab + 4]
        src_q, dst_q, rows, agg, tab_sh = refs[ntab + 4:ntab + 9]
        sems = refs[ntab + 9:]
        gsems = sems[:_NBUF_S]
        ssems = sems[_NBUF_S:]
        c = lax.axis_index("c")
        s = lax.axis_index("s")
        t_rows = N_NODES // _NS
        pltpu.sync_copy(zrows, agg.at[pl.ds(s * _ROWS_PER_TILE, _ROWS_PER_TILE)])
        for t in range(ntab):
            pltpu.sync_copy(
                tables[t].at[pl.ds(c * N_NODES + s * t_rows, t_rows)],
                tab_sh.at[pl.ds(s * t_rows, t_rows)])
            plsc.subcore_barrier()
            for q in range(_NCH // _QCH):
                base_row = s * _NCH + q * _QCH
                pltpu.sync_copy(src2d.at[pl.ds(base_row, _QCH)], src_q)
                pltpu.sync_copy(dstix.at[pl.ds(base_row, _QCH)], dst_q)
                _edge_pass(tab_sh, src_q, dst_q, rows, agg, gsems, ssems,
                           _NBUF_S, _QCH)
            plsc.subcore_barrier()
            _writeback_and_zero(out.at[t, c], zrows, rows, agg, c, s,
                                zero_after=(t + 1 < ntab), sem=sems[0])
    return body


def _segsum_multi(tables, src2d, dst2d, zrows):
    ntab = len(tables)
    f = pl.kernel(
        _make_multi_body(ntab),
        out_type=jax.ShapeDtypeStruct((ntab, _NC, _AGG_ROWS, _HALF), _F32),
        mesh=_sc_mesh(),
        scratch_types=(
            [
                pltpu.VMEM((_QCH, _CHUNK), jnp.int32),
                pltpu.VMEM((_QCH, _CHUNK), jnp.int32),
                pltpu.VMEM((_NBUF_S, _CHUNK, _HALF), _F32),
                pltpu.VMEM_SHARED((_AGG_ROWS, _HALF), _F32),
                pltpu.VMEM_SHARED((N_NODES, _HALF), _F32),
            ]
            + [pltpu.SemaphoreType.DMA] * (2 * _NBUF_S)
        ),
        compiler_params=pltpu.CompilerParams(use_tc_tiling_on_sc=False),
    )
    return f(*tables, src2d, dst2d, zrows)


def _segsum2k_body(table, srcix2k, dstix, zrows, out,
                   src_all, dst_all, rows, agg, *sems):
    gsems = sems[:_NBUF]
    ssems = sems[_NBUF:]
    c = lax.axis_index("c")
    s = lax.axis_index("s")
    base_row = s * _NCH
    pltpu.sync_copy(zrows, agg.at[pl.ds(s * _ROWS_PER_TILE, _ROWS_PER_TILE)])
    pltpu.sync_copy(dstix.at[pl.ds(base_row, _NCH)], dst_all)
    for k in range(2):
        pltpu.sync_copy(srcix2k.at[k, c, pl.ds(base_row, _NCH)], src_all)
        plsc.subcore_barrier()
        _edge_pass(table, src_all, dst_all, rows, agg, gsems, ssems,
                   _NBUF, _NCH)
        plsc.subcore_barrier()
        _writeback_and_zero(out.at[k, c], zrows, rows, agg, c, s,
                            zero_after=(k == 0), sem=sems[0])


def _segsum2k(table_flat, srcix2k, dst2d, zrows):
    f = pl.kernel(
        _segsum2k_body,
        out_type=jax.ShapeDtypeStruct((2, _NC, _AGG_ROWS, _HALF), _F32),
        mesh=_sc_mesh(),
        scratch_types=(
            [
                pltpu.VMEM((_NCH, _CHUNK), jnp.int32),
                pltpu.VMEM((_NCH, _CHUNK), jnp.int32),
                pltpu.VMEM((_NBUF, _CHUNK, _HALF), _F32),
                pltpu.VMEM_SHARED((_AGG_ROWS, _HALF), _F32),
            ]
            + [pltpu.SemaphoreType.DMA] * (2 * _NBUF)
        ),
        compiler_params=pltpu.CompilerParams(use_tc_tiling_on_sc=False),
    )
    return f(table_flat, srcix2k, dst2d, zrows)


def _gidx_body(srcf, dstf, topi2, out, srcb, dstb, topi_v, gbuf, sem):
    c = lax.axis_index("c")
    s = lax.axis_index("s")
    base = s * _E_PER_T
    pltpu.sync_copy(srcf.at[pl.ds(base, _E_PER_T)], srcb)
    pltpu.sync_copy(dstf.at[pl.ds(base, _E_PER_T)], dstb)
    pltpu.sync_copy(topi2, topi_v)
    coff = c * (N_EXPERTS * N_NODES)
    nrow = _AGG_ROWS // _CHUNK
    for k in range(2):
        def step(i, carry):
            sv = srcb[pl.ds(i * 16, 16)]
            dv = dstb[pl.ds(i * 16, 16)]
            ev = plsc.load_gather(
                topi_v, [k * nrow + lax.shift_right_logical(dv, 7),
                         lax.bitwise_and(dv, 127)])
            gbuf[pl.ds(i * 16, 16)] = ev * N_NODES + sv + coff
            return carry
        lax.fori_loop(0, _E_PER_T // 16, step, 0)
        pltpu.sync_copy(gbuf, out.at[k, c, pl.ds(base, _E_PER_T)])


def _gidx(srcf, dstf, topi2):
    f = pl.kernel(
        _gidx_body,
        out_type=jax.ShapeDtypeStruct((2, _NC, _E_PAD), jnp.int32),
        mesh=_sc_mesh(),
        scratch_types=[
            pltpu.VMEM((_E_PER_T,), jnp.int32),
            pltpu.VMEM((_E_PER_T,), jnp.int32),
            pltpu.VMEM((2 * (_AGG_ROWS // _CHUNK), _CHUNK), jnp.int32),
            pltpu.VMEM((_E_PER_T,), jnp.int32),
            pltpu.SemaphoreType.DMA,
        ],
        compiler_params=pltpu.CompilerParams(needs_layout_passes=False),
    )
    return f(srcf, dstf, topi2)


_BLK = 1000


def _stats_kernel(batch_ref, src_ref, stats_ref):
    b = batch_ref[...]
    srows = src_ref[...].astype(_F32)
    gids = lax.broadcasted_iota(jnp.int32, (1, N_GRAPHS), 1)
    oh = (b == gids).astype(_F32)
    n_per = jnp.sum(oh, axis=0, keepdims=True)
    ii = lax.broadcasted_iota(jnp.int32, (N_GRAPHS, N_GRAPHS), 0)
    jj = lax.broadcasted_iota(jnp.int32, (N_GRAPHS, N_GRAPHS), 1)
    tri = (ii <= jj).astype(_F32)
    bounds = lax.dot_general(n_per, tri, (((1,), (0,)), ((), ())),
                             preferred_element_type=_F32, precision=_HIGH)
    cnt_prev = jnp.zeros((1, 1), _F32)
    e_list = []
    for g in range(N_GRAPHS):
        cnt = jnp.sum(jnp.where(srows < bounds[:, g:g + 1], 1.0, 0.0)).reshape(1, 1)
        e_list.append(cnt - cnt_prev)
        cnt_prev = cnt
    e_per = jnp.concatenate(e_list, axis=1)
    n = jnp.maximum(n_per, 1.0)
    e = jnp.maximum(e_per, 0.0)
    log_n = jnp.log(n)
    log_e = jnp.log1p(e)
    log_n_norm = ((log_n - jnp.min(log_n))
                  / (jnp.max(log_n) - jnp.min(log_n) + 1e-6))
    def _std(v):
        m = jnp.mean(v)
        sd = jnp.sqrt(jnp.mean((v - m) ** 2))
        return (v - m) / (sd + 1e-6)
    stats_ref[...] = jnp.concatenate(
        [_std(log_n), _std(log_e), log_n_norm], axis=0)


def _stats_call(batch1, src2d_real):
    return pl.pallas_call(
        _stats_kernel,
        out_shape=jax.ShapeDtypeStruct((3, N_GRAPHS), _F32),
    )(batch1, src2d_real)


def _router_kernel(x_ref, b_ref, stats_ref, ew1_ref, eb1_ref, ew2_ref, eb2_ref,
                   rw1h_ref, rw1f_ref, rb1_ref, lg_ref, lb_ref, rw2_ref, rb2_ref,
                   cen_ref, h_ref, hs_ref, w_ref, ti_ref):
    xs = x_ref[...][:, 4:10]
    h1 = jax.nn.relu(
        lax.dot_general(xs, ew1_ref[...], (((1,), (1,)), ((), ())),
                        preferred_element_type=_F32, precision=_HIGH)
        + eb1_ref[...])
    h = lax.dot_general(h1, ew2_ref[...], (((1,), (1,)), ((), ())),
                        preferred_element_type=_F32, precision=_HIGH) + eb2_ref[...]
    h_ref[...] = h
    hs_ref[0] = h[:, :_HALF]
    hs_ref[1] = h[:, _HALF:]
    gids = lax.broadcasted_iota(jnp.int32, (1, N_GRAPHS), 1)
    oh = (b_ref[...] == gids).astype(_F32)
    nf = lax.dot_general(oh, stats_ref[...], (((1,), (1,)), ((), ())),
                         preferred_element_type=_F32, precision=_HIGH)
    r = (lax.dot_general(h, rw1h_ref[...], (((1,), (1,)), ((), ())),
                         preferred_element_type=_F32, precision=_HIGH)
         + lax.dot_general(nf[:, 0:2], rw1f_ref[...], (((1,), (1,)), ((), ())),
                           preferred_element_type=_F32, precision=_HIGH)
         + rb1_ref[...])
    mu = jnp.mean(r, axis=-1, keepdims=True)
    var = jnp.mean((r - mu) ** 2, axis=-1, keepdims=True)
    r = (r - mu) * lax.rsqrt(var + 1e-5) * lg_ref[...] + lb_ref[...]
    r = jax.nn.relu(r)
    learned = lax.dot_general(r, rw2_ref[...], (((1,), (1,)), ((), ())),
                              preferred_element_type=_F32, precision=_HIGH) + rb2_ref[...]
    prior = -((nf[:, 2:3] - cen_ref[...]) ** 2)
    logits = 0.65 * learned + 0.35 * prior
    m = jnp.max(logits, axis=-1, keepdims=True)
    ex = jnp.exp(logits - m)
    p = ex / jnp.sum(ex, axis=-1, keepdims=True)
    ids = lax.broadcasted_iota(jnp.int32, p.shape, 1)
    m1 = jnp.max(p, axis=-1, keepdims=True)
    i1 = jnp.min(jnp.where(p == m1, ids, N_EXPERTS), axis=-1, keepdims=True)
    p2 = jnp.where(ids == i1, -1.0, p)
    m2 = jnp.max(p2, axis=-1, keepdims=True)
    i2 = jnp.min(jnp.where(p2 == m2, ids, N_EXPERTS), axis=-1, keepdims=True)
    tot = m1 + m2
    w_ref[...] = jnp.concatenate([m1 / tot, m2 / tot], axis=-1)
    ti_ref[...] = jnp.concatenate([i1, i2], axis=-1)


def _router_call(x, batch1, stats, enc_w1, enc_b1, enc_w2, enc_b2,
                 rw1h, rw1f, rb1, ln_g, ln_b, rw2, rb2, cen):
    grid = N_NODES // _BLK
    bs_in = [
        pl.BlockSpec((_BLK, 16), lambda i: (i, 0)),
        pl.BlockSpec((_BLK, 1), lambda i: (i, 0)),
        pl.BlockSpec((3, N_GRAPHS), lambda i: (0, 0)),
        pl.BlockSpec((HID, 6), lambda i: (0, 0)),
        pl.BlockSpec((1, HID), lambda i: (0, 0)),
        pl.BlockSpec((HID, HID), lambda i: (0, 0)),
        pl.BlockSpec((1, HID), lambda i: (0, 0)),
        pl.BlockSpec((HID, HID), lambda i: (0, 0)),
        pl.BlockSpec((HID, 2), lambda i: (0, 0)),
        pl.BlockSpec((1, HID), lambda i: (0, 0)),
        pl.BlockSpec((1, HID), lambda i: (0, 0)),
        pl.BlockSpec((1, HID), lambda i: (0, 0)),
        pl.BlockSpec((N_EXPERTS, HID), lambda i: (0, 0)),
        pl.BlockSpec((1, N_EXPERTS), lambda i: (0, 0)),
        pl.BlockSpec((1, N_EXPERTS), lambda i: (0, 0)),
    ]
    bs_out = [
        pl.BlockSpec((_BLK, HID), lambda i: (i, 0)),
        pl.BlockSpec((_NC, _BLK, _HALF), lambda i: (0, i, 0)),
        pl.BlockSpec((_BLK, 2), lambda i: (i, 0)),
        pl.BlockSpec((_BLK, 2), lambda i: (i, 0)),
    ]
    return pl.pallas_call(
        _router_kernel,
        grid=(grid,),
        in_specs=bs_in,
        out_specs=bs_out,
        out_shape=[
            jax.ShapeDtypeStruct((N_NODES, HID), _F32),
            jax.ShapeDtypeStruct((_NC, N_NODES, _HALF), _F32),
            jax.ShapeDtypeStruct((N_NODES, 2), _F32),
            jax.ShapeDtypeStruct((N_NODES, 2), jnp.int32),
        ],
    )(x, batch1, stats, enc_w1, enc_b1, enc_w2, enc_b2,
      rw1h, rw1f, rb1, ln_g, ln_b, rw2, rb2, cen)


def _cat_agg(aggp_ref):
    return jnp.concatenate([aggp_ref[0], aggp_ref[1]], axis=-1)


def _layer0_kernel(aggp_ref, h_ref, wr_ref, wo_ref, b_ref, out_ref, osp_ref):
    agg = _cat_agg(aggp_ref)
    h = h_ref[...]
    for e in range(N_EXPERTS):
        y = (lax.dot_general(agg, wr_ref[e], (((1,), (1,)), ((), ())),
                             preferred_element_type=_F32, precision=_HIGH)
             + lax.dot_general(h, wo_ref[e], (((1,), (1,)), ((), ())),
                               preferred_element_type=_F32, precision=_HIGH)
             + b_ref[e:e + 1, :])
        y = jax.nn.relu(y)
        out_ref[e] = y
        osp_ref[e, 0] = y[:, :_HALF]
        osp_ref[e, 1] = y[:, _HALF:]


def _layer0_call(aggp, h, wr0, wo0, b0):
    grid = N_NODES // _BLK
    return pl.pallas_call(
        _layer0_kernel,
        grid=(grid,),
        in_specs=[
            pl.BlockSpec((_NC, _BLK, _HALF), lambda i: (0, i, 0)),
            pl.BlockSpec((_BLK, HID), lambda i: (i, 0)),
            pl.BlockSpec((N_EXPERTS, HID, HID), lambda i: (0, 0, 0)),
            pl.BlockSpec((N_EXPERTS, HID, HID), lambda i: (0, 0, 0)),
            pl.BlockSpec((N_EXPERTS, HID), lambda i: (0, 0)),
        ],
        out_specs=[
            pl.BlockSpec((N_EXPERTS, _BLK, HID), lambda i: (0, i, 0)),
            pl.BlockSpec((N_EXPERTS, _NC, _BLK, _HALF), lambda i: (0, 0, i, 0)),
        ],
        out_shape=[
            jax.ShapeDtypeStruct((N_EXPERTS, N_NODES, HID), _F32),
            jax.ShapeDtypeStruct((N_EXPERTS, _NC, N_NODES, _HALF), _F32),
        ],
    )(aggp, h, wr0, wo0, b0)


def _layer1_kernel(*refs):
    aggp_refs = refs[:N_EXPERTS]
    y0_ref, wr_ref, wo_ref, b_ref, out_ref, osp_ref = refs[N_EXPERTS:]
    for e in range(N_EXPERTS):
        agg = _cat_agg(aggp_refs[e])
        y = (lax.dot_general(agg, wr_ref[e], (((1,), (1,)), ((), ())),
                             preferred_element_type=_F32, precision=_HIGH)
             + lax.dot_general(y0_ref[e], wo_ref[e], (((1,), (1,)), ((), ())),
                               preferred_element_type=_F32, precision=_HIGH)
             + b_ref[e:e + 1, :])
        y = jax.nn.relu(y)
        out_ref[e] = y
        osp_ref[0, e] = y[:, :_HALF]
        osp_ref[1, e] = y[:, _HALF:]


def _layer1_call(aggp_list, y0, wr1, wo1, b1):
    grid = N_NODES // _BLK
    in_specs = (
        [pl.BlockSpec((_NC, _BLK, _HALF), lambda i: (0, i, 0))] * N_EXPERTS
        + [
            pl.BlockSpec((N_EXPERTS, _BLK, HID), lambda i: (0, i, 0)),
            pl.BlockSpec((N_EXPERTS, HID, HID), lambda i: (0, 0, 0)),
            pl.BlockSpec((N_EXPERTS, HID, HID), lambda i: (0, 0, 0)),
            pl.BlockSpec((N_EXPERTS, HID), lambda i: (0, 0)),
        ]
    )
    return pl.pallas_call(
        _layer1_kernel,
        grid=(grid,),
        in_specs=in_specs,
        out_specs=[
            pl.BlockSpec((N_EXPERTS, _BLK, HID), lambda i: (0, i, 0)),
            pl.BlockSpec((_NC, N_EXPERTS, _BLK, _HALF), lambda i: (0, 0, i, 0)),
        ],
        out_shape=[
            jax.ShapeDtypeStruct((N_EXPERTS, N_NODES, HID), _F32),
            jax.ShapeDtypeStruct((_NC, N_EXPERTS, N_NODES, _HALF), _F32),
        ],
    )(*aggp_list, y0, wr1, wo1, b1)


def _final_kernel(a0p_ref, a1p_ref, y1_ref, w_ref, ti_ref, wr2_ref, wo2_ref,
                  b2_ref, out_ref):
    a0 = _cat_agg(a0p_ref)
    a1 = _cat_agg(a1p_ref)
    w = w_ref[...]
    ti = ti_ref[...]
    acc = jnp.zeros((a0.shape[0], OUT), _F32)
    for e in range(N_EXPERTS):
        r0 = lax.dot_general(a0, wr2_ref[e], (((1,), (1,)), ((), ())),
                             preferred_element_type=_F32, precision=_HIGH)
        r1 = lax.dot_general(a1, wr2_ref[e], (((1,), (1,)), ((), ())),
                             preferred_element_type=_F32, precision=_HIGH)
        se = lax.dot_general(y1_ref[e], wo2_ref[e], (((1,), (1,)), ((), ())),
                             preferred_element_type=_F32, precision=_HIGH)
        base = se + b2_ref[e:e + 1, :]
        sel0 = (ti[:, 0:1] == e).astype(_F32)
        sel1 = (ti[:, 1:2] == e).astype(_F32)
        acc = acc + w[:, 0:1] * sel0 * (r0 + base) + w[:, 1:2] * sel1 * (r1 + base)
    out_ref[...] = acc


def _final_call(a0p, a1p, y1, wts, topi, wr2, wo2, b2):
    grid = N_NODES // _BLK
    return pl.pallas_call(
        _final_kernel,
        grid=(grid,),
        in_specs=[
            pl.BlockSpec((_NC, _BLK, _HALF), lambda i: (0, i, 0)),
            pl.BlockSpec((_NC, _BLK, _HALF), lambda i: (0, i, 0)),
            pl.BlockSpec((N_EXPERTS, _BLK, HID), lambda i: (0, i, 0)),
            pl.BlockSpec((_BLK, 2), lambda i: (i, 0)),
            pl.BlockSpec((_BLK, 2), lambda i: (i, 0)),
            pl.BlockSpec((N_EXPERTS, OUT, HID), lambda i: (0, 0, 0)),
            pl.BlockSpec((N_EXPERTS, OUT, HID), lambda i: (0, 0, 0)),
            pl.BlockSpec((N_EXPERTS, OUT), lambda i: (0, 0)),
        ],
        out_specs=pl.BlockSpec((_BLK, OUT), lambda i: (i, 0)),
        out_shape=jax.ShapeDtypeStruct((N_NODES, OUT), _F32),
    )(a0p, a1p, y1, wts, topi, wr2, wo2, b2)


def kernel(x, edge_index, batch, enc_w1, enc_b1, enc_w2, enc_b2,
           rout_w1, rout_b1, ln_g, ln_b, rout_w2, rout_b2, centers,
           ewr0, ewo0, eb0, ewr1, ewo1, eb1, ewr2, ewo2, eb2):
    src = edge_index[0]
    dst = edge_index[1]
    srcf = jnp.pad(src, (0, _E_PAD - N_EDGES))
    dstf = jnp.pad(dst, (0, _E_PAD - N_EDGES), constant_values=_DUMP)
    dst2d = dstf.reshape(_E_PAD // _CHUNK, _CHUNK)
    src2d = srcf.reshape(_E_PAD // _CHUNK, _CHUNK)
    zrows = jnp.zeros((_ROWS_PER_TILE, _HALF), _F32)
    batch1 = batch[:, None]
    src2d_real = src.reshape(N_EDGES // _CHUNK, _CHUNK)

    stats = _stats_call(batch1, src2d_real)
    h, hsplit, wts, topi = _router_call(
        x, batch1, stats, enc_w1, enc_b1[None, :], enc_w2, enc_b2[None, :],
        rout_w1[:, :HID], rout_w1[:, HID:], rout_b1[None, :],
        ln_g[None, :], ln_b[None, :], rout_w2, rout_b2[None, :],
        centers[None, :])

    agg0 = _segsum_multi([hsplit.reshape(_NC * N_NODES, _HALF)],
                         src2d, dst2d, zrows)[0]
    y0, y0split = _layer0_call(agg0, h, ewr0, ewo0, eb0)

    agg1 = []
    for e0 in range(0, N_EXPERTS, 4):
        quad = _segsum_multi(
            [y0split[e0 + i].reshape(_NC * N_NODES, _HALF) for i in range(4)],
            src2d, dst2d, zrows)
        agg1 += [quad[i] for i in range(4)]
    y1, y1split = _layer1_call(agg1, y0, ewr1, ewo1, eb1)

    topi2 = jnp.pad(topi.T, ((0, 0), (0, _AGG_ROWS - N_NODES))).reshape(
        2 * (_AGG_ROWS // _CHUNK), _CHUNK)
    gix = _gidx(srcf, dstf, topi2)
    y1flat = y1split.reshape(_NC * N_EXPERTS * N_NODES, _HALF)
    akp = _segsum2k(y1flat,
                    gix.reshape(2, _NC, _E_PAD // _CHUNK, _CHUNK),
                    dst2d, zrows)

    return _final_call(akp[0], akp[1], y1, wts, topi, ewr2, ewo2, eb2)


# speedup vs baseline: 1.0198x; 1.0198x over previous
import jax
import jax.numpy as jnp
from jax import lax
from jax.experimental import pallas as pl
from jax.experimental.pallas import tpu as pltpu
from jax.experimental.pallas import tpu_sc as plsc

N_NODES = 10000
N_EDGES = 320000
N_GRAPHS = 16
N_EXPERTS = 8
HID = 128
OUT = 6

_NC = 2
_NS = 16

_HALF = HID // 2
_CHUNK = 128
_E_PER_T = 20480
_E_PAD = _NS * _E_PER_T
_NCH = _E_PER_T // _CHUNK
_NBUF = 6
_NBUF_S = 3
_QCH = 40

_AGG_ROWS = 10240
_ROWS_PER_TILE = _AGG_ROWS // _NS
_DUMP = _AGG_ROWS - 1

_F32 = jnp.float32
_HIGH = jax.lax.Precision.HIGHEST


def _sc_mesh():
    return plsc.VectorSubcoreMesh(core_axis_name="c", subcore_axis_name="s")


def _edge_pass(table, src_all, dst_all, rows, agg, gsems, ssems, nbuf, nch):
    g = [None] * nbuf
    sc = [None] * nbuf

    def start_gather(j):
        b = j % nbuf
        g[b] = pltpu.async_copy(table.at[src_all.at[j]], rows.at[b], gsems[b])

    def start_scatter(j):
        b = j % nbuf
        sc[b] = pltpu.async_copy(rows.at[b], agg.at[dst_all.at[j]], ssems[b],
                                 add=True)

    for j in range(nbuf - 1):
        start_gather(j)
    for j in range(nch):
        b = j % nbuf
        g[b].wait()
        start_scatter(j)
        jn = j + nbuf - 1
        if jn < nch:
            bn = jn % nbuf
            if sc[bn] is not None:
                sc[bn].wait()
            start_gather(jn)
    for j in range(nch - nbuf, nch):
        if sc[j % nbuf] is not None:
            sc[j % nbuf].wait()


def _writeback_and_zero(out_slot, zrows, rows, agg, c, s, zero_after, sem):
    h = pltpu.async_copy(
        agg.at[pl.ds(s * _ROWS_PER_TILE, _ROWS_PER_TILE)],
        out_slot.at[pl.ds(s * _ROWS_PER_TILE, _ROWS_PER_TILE)], sem)
    h.wait()
    if zero_after:
        pltpu.sync_copy(
            zrows, agg.at[pl.ds(s * _ROWS_PER_TILE, _ROWS_PER_TILE)])
        plsc.subcore_barrier()


def _make_multi_body(ntab):
    def body(*refs):
        tables = refs[:ntab]
        src2d, dstix, zrows, out = refs[ntab:ntab + 4]
        src_q, dst_q, rows, agg, tab_sh = refs[ntab + 4:ntab + 9]
        sems = refs[ntab + 9:]
        gsems = sems[:_NBUF_S]
        ssems = sems[_NBUF_S:]
        c = lax.axis_index("c")
        s = lax.axis_index("s")
        t_rows = N_NODES // _NS
        pltpu.sync_copy(zrows, agg.at[pl.ds(s * _ROWS_PER_TILE, _ROWS_PER_TILE)])
        for t in range(ntab):
            pltpu.sync_copy(
                tables[t].at[pl.ds(c * N_NODES + s * t_rows, t_rows)],
                tab_sh.at[pl.ds(s * t_rows, t_rows)])
            plsc.subcore_barrier()
            for q in range(_NCH // _QCH):
                base_row = s * _NCH + q * _QCH
                pltpu.sync_copy(src2d.at[pl.ds(base_row, _QCH)], src_q)
                pltpu.sync_copy(dstix.at[pl.ds(base_row, _QCH)], dst_q)
                _edge_pass(tab_sh, src_q, dst_q, rows, agg, gsems, ssems,
                           _NBUF_S, _QCH)
            plsc.subcore_barrier()
            _writeback_and_zero(out.at[t, c], zrows, rows, agg, c, s,
                                zero_after=(t + 1 < ntab), sem=sems[0])
    return body


def _segsum_multi(tables, src2d, dst2d, zrows):
    ntab = len(tables)
    f = pl.kernel(
        _make_multi_body(ntab),
        out_type=jax.ShapeDtypeStruct((ntab, _NC, _AGG_ROWS, _HALF), _F32),
        mesh=_sc_mesh(),
        scratch_types=(
            [
                pltpu.VMEM((_QCH, _CHUNK), jnp.int32),
                pltpu.VMEM((_QCH, _CHUNK), jnp.int32),
                pltpu.VMEM((_NBUF_S, _CHUNK, _HALF), _F32),
                pltpu.VMEM_SHARED((_AGG_ROWS, _HALF), _F32),
                pltpu.VMEM_SHARED((N_NODES, _HALF), _F32),
            ]
            + [pltpu.SemaphoreType.DMA] * (2 * _NBUF_S)
        ),
        compiler_params=pltpu.CompilerParams(use_tc_tiling_on_sc=False),
    )
    return f(*tables, src2d, dst2d, zrows)


def _segsum2k_body(table, srcix2k, dstix, zrows, out,
                   src_all, dst_all, rows, agg, *sems):
    gsems = sems[:_NBUF]
    ssems = sems[_NBUF:]
    c = lax.axis_index("c")
    s = lax.axis_index("s")
    base_row = s * _NCH
    pltpu.sync_copy(zrows, agg.at[pl.ds(s * _ROWS_PER_TILE, _ROWS_PER_TILE)])
    pltpu.sync_copy(dstix.at[pl.ds(base_row, _NCH)], dst_all)
    for k in range(2):
        pltpu.sync_copy(srcix2k.at[k, c, pl.ds(base_row, _NCH)], src_all)
        plsc.subcore_barrier()
        _edge_pass(table, src_all, dst_all, rows, agg, gsems, ssems,
                   _NBUF, _NCH)
        plsc.subcore_barrier()
        _writeback_and_zero(out.at[k, c], zrows, rows, agg, c, s,
                            zero_after=(k == 0), sem=sems[0])


def _segsum2k(table_flat, srcix2k, dst2d, zrows):
    f = pl.kernel(
        _segsum2k_body,
        out_type=jax.ShapeDtypeStruct((2, _NC, _AGG_ROWS, _HALF), _F32),
        mesh=_sc_mesh(),
        scratch_types=(
            [
                pltpu.VMEM((_NCH, _CHUNK), jnp.int32),
                pltpu.VMEM((_NCH, _CHUNK), jnp.int32),
                pltpu.VMEM((_NBUF, _CHUNK, _HALF), _F32),
                pltpu.VMEM_SHARED((_AGG_ROWS, _HALF), _F32),
            ]
            + [pltpu.SemaphoreType.DMA] * (2 * _NBUF)
        ),
        compiler_params=pltpu.CompilerParams(use_tc_tiling_on_sc=False),
    )
    return f(table_flat, srcix2k, dst2d, zrows)


def _gidx_body(srcf, dstf, topi2, out, srcb, dstb, topi_v, gbuf, sem):
    c = lax.axis_index("c")
    s = lax.axis_index("s")
    base = s * _E_PER_T
    pltpu.sync_copy(srcf.at[pl.ds(base, _E_PER_T)], srcb)
    pltpu.sync_copy(dstf.at[pl.ds(base, _E_PER_T)], dstb)
    pltpu.sync_copy(topi2, topi_v)
    coff = c * (N_EXPERTS * N_NODES)
    nrow = _AGG_ROWS // _CHUNK
    for k in range(2):
        def step(i, carry):
            sv = srcb[pl.ds(i * 16, 16)]
            dv = dstb[pl.ds(i * 16, 16)]
            ev = plsc.load_gather(
                topi_v, [k * nrow + lax.shift_right_logical(dv, 7),
                         lax.bitwise_and(dv, 127)])
            gbuf[pl.ds(i * 16, 16)] = ev * N_NODES + sv + coff
            return carry
        lax.fori_loop(0, _E_PER_T // 16, step, 0)
        pltpu.sync_copy(gbuf, out.at[k, c, pl.ds(base, _E_PER_T)])


def _gidx(srcf, dstf, topi2):
    f = pl.kernel(
        _gidx_body,
        out_type=jax.ShapeDtypeStruct((2, _NC, _E_PAD), jnp.int32),
        mesh=_sc_mesh(),
        scratch_types=[
            pltpu.VMEM((_E_PER_T,), jnp.int32),
            pltpu.VMEM((_E_PER_T,), jnp.int32),
            pltpu.VMEM((2 * (_AGG_ROWS // _CHUNK), _CHUNK), jnp.int32),
            pltpu.VMEM((_E_PER_T,), jnp.int32),
            pltpu.SemaphoreType.DMA,
        ],
        compiler_params=pltpu.CompilerParams(needs_layout_passes=False),
    )
    return f(srcf, dstf, topi2)


_BLK = 1000


def _stats_kernel(batch_ref, src_ref, stats_ref):
    b = batch_ref[...]
    srows = src_ref[...].astype(_F32)
    gids = lax.broadcasted_iota(jnp.int32, (1, N_GRAPHS), 1)
    oh = (b == gids).astype(_F32)
    n_per = jnp.sum(oh, axis=0, keepdims=True)
    ii = lax.broadcasted_iota(jnp.int32, (N_GRAPHS, N_GRAPHS), 0)
    jj = lax.broadcasted_iota(jnp.int32, (N_GRAPHS, N_GRAPHS), 1)
    tri = (ii <= jj).astype(_F32)
    bounds = lax.dot_general(n_per, tri, (((1,), (0,)), ((), ())),
                             preferred_element_type=_F32, precision=_HIGH)
    cnt_prev = jnp.zeros((1, 1), _F32)
    e_list = []
    for g in range(N_GRAPHS):
        cnt = jnp.sum(jnp.where(srows < bounds[:, g:g + 1], 1.0, 0.0)).reshape(1, 1)
        e_list.append(cnt - cnt_prev)
        cnt_prev = cnt
    e_per = jnp.concatenate(e_list, axis=1)
    n = jnp.maximum(n_per, 1.0)
    e = jnp.maximum(e_per, 0.0)
    log_n = jnp.log(n)
    log_e = jnp.log1p(e)
    log_n_norm = ((log_n - jnp.min(log_n))
                  / (jnp.max(log_n) - jnp.min(log_n) + 1e-6))
    def _std(v):
        m = jnp.mean(v)
        sd = jnp.sqrt(jnp.mean((v - m) ** 2))
        return (v - m) / (sd + 1e-6)
    stats_ref[...] = jnp.concatenate(
        [_std(log_n), _std(log_e), log_n_norm], axis=0)


def _stats_call(batch1, src2d_real):
    return pl.pallas_call(
        _stats_kernel,
        out_shape=jax.ShapeDtypeStruct((3, N_GRAPHS), _F32),
    )(batch1, src2d_real)


def _router_kernel(x_ref, b_ref, stats_ref, ew1_ref, eb1_ref, ew2_ref, eb2_ref,
                   rw1h_ref, rw1f_ref, rb1_ref, lg_ref, lb_ref, rw2_ref, rb2_ref,
                   cen_ref, h_ref, hs_ref, w_ref, ti_ref):
    xs = x_ref[...][:, 4:10]
    h1 = jax.nn.relu(
        lax.dot_general(xs, ew1_ref[...], (((1,), (1,)), ((), ())),
                        preferred_element_type=_F32, precision=_HIGH)
        + eb1_ref[...])
    h = lax.dot_general(h1, ew2_ref[...], (((1,), (1,)), ((), ())),
                        preferred_element_type=_F32, precision=_HIGH) + eb2_ref[...]
    h_ref[...] = h
    hs_ref[0] = h[:, :_HALF]
    hs_ref[1] = h[:, _HALF:]
    gids = lax.broadcasted_iota(jnp.int32, (1, N_GRAPHS), 1)
    oh = (b_ref[...] == gids).astype(_F32)
    nf = lax.dot_general(oh, stats_ref[...], (((1,), (1,)), ((), ())),
                         preferred_element_type=_F32, precision=_HIGH)
    r = (lax.dot_general(h, rw1h_ref[...], (((1,), (1,)), ((), ())),
                         preferred_element_type=_F32, precision=_HIGH)
         + lax.dot_general(nf[:, 0:2], rw1f_ref[...], (((1,), (1,)), ((), ())),
                           preferred_element_type=_F32, precision=_HIGH)
         + rb1_ref[...])
    mu = jnp.mean(r, axis=-1, keepdims=True)
    var = jnp.mean((r - mu) ** 2, axis=-1, keepdims=True)
    r = (r - mu) * lax.rsqrt(var + 1e-5) * lg_ref[...] + lb_ref[...]
    r = jax.nn.relu(r)
    learned = lax.dot_general(r, rw2_ref[...], (((1,), (1,)), ((), ())),
                              preferred_element_type=_F32, precision=_HIGH) + rb2_ref[...]
    prior = -((nf[:, 2:3] - cen_ref[...]) ** 2)
    logits = 0.65 * learned + 0.35 * prior
    m = jnp.max(logits, axis=-1, keepdims=True)
    ex = jnp.exp(logits - m)
    p = ex / jnp.sum(ex, axis=-1, keepdims=True)
    ids = lax.broadcasted_iota(jnp.int32, p.shape, 1)
    m1 = jnp.max(p, axis=-1, keepdims=True)
    i1 = jnp.min(jnp.where(p == m1, ids, N_EXPERTS), axis=-1, keepdims=True)
    p2 = jnp.where(ids == i1, -1.0, p)
    m2 = jnp.max(p2, axis=-1, keepdims=True)
    i2 = jnp.min(jnp.where(p2 == m2, ids, N_EXPERTS), axis=-1, keepdims=True)
    tot = m1 + m2
    w_ref[...] = jnp.concatenate([m1 / tot, m2 / tot], axis=-1)
    ti_ref[...] = jnp.concatenate([i1, i2], axis=-1)


def _router_call(x, batch1, stats, enc_w1, enc_b1, enc_w2, enc_b2,
                 rw1h, rw1f, rb1, ln_g, ln_b, rw2, rb2, cen):
    grid = N_NODES // _BLK
    bs_in = [
        pl.BlockSpec((_BLK, 16), lambda i: (i, 0)),
        pl.BlockSpec((_BLK, 1), lambda i: (i, 0)),
        pl.BlockSpec((3, N_GRAPHS), lambda i: (0, 0)),
        pl.BlockSpec((HID, 6), lambda i: (0, 0)),
        pl.BlockSpec((1, HID), lambda i: (0, 0)),
        pl.BlockSpec((HID, HID), lambda i: (0, 0)),
        pl.BlockSpec((1, HID), lambda i: (0, 0)),
        pl.BlockSpec((HID, HID), lambda i: (0, 0)),
        pl.BlockSpec((HID, 2), lambda i: (0, 0)),
        pl.BlockSpec((1, HID), lambda i: (0, 0)),
        pl.BlockSpec((1, HID), lambda i: (0, 0)),
        pl.BlockSpec((1, HID), lambda i: (0, 0)),
        pl.BlockSpec((N_EXPERTS, HID), lambda i: (0, 0)),
        pl.BlockSpec((1, N_EXPERTS), lambda i: (0, 0)),
        pl.BlockSpec((1, N_EXPERTS), lambda i: (0, 0)),
    ]
    bs_out = [
        pl.BlockSpec((_BLK, HID), lambda i: (i, 0)),
        pl.BlockSpec((_NC, _BLK, _HALF), lambda i: (0, i, 0)),
        pl.BlockSpec((_BLK, 2), lambda i: (i, 0)),
        pl.BlockSpec((_BLK, 2), lambda i: (i, 0)),
    ]
    return pl.pallas_call(
        _router_kernel,
        grid=(grid,),
        in_specs=bs_in,
        out_specs=bs_out,
        out_shape=[
            jax.ShapeDtypeStruct((N_NODES, HID), _F32),
            jax.ShapeDtypeStruct((_NC, N_NODES, _HALF), _F32),
            jax.ShapeDtypeStruct((N_NODES, 2), _F32),
            jax.ShapeDtypeStruct((N_NODES, 2), jnp.int32),
        ],
    )(x, batch1, stats, enc_w1, enc_b1, enc_w2, enc_b2,
      rw1h, rw1f, rb1, ln_g, ln_b, rw2, rb2, cen)


def _cat_agg(aggp_ref):
    return jnp.concatenate([aggp_ref[0], aggp_ref[1]], axis=-1)


def _split_bf16(a):
    return a


def _dot3(a, b):
    dn = (((1,), (1,)), ((), ()))
    return lax.dot_general(a, b, dn, preferred_element_type=_F32,
                           precision=_HIGH)


def _layer0_kernel(aggp_ref, h_ref, wr_ref, wo_ref, b_ref, out_ref, osp_ref):
    agg2 = _split_bf16(_cat_agg(aggp_ref))
    h2 = _split_bf16(h_ref[...])
    for e in range(N_EXPERTS):
        y = (_dot3(agg2, _split_bf16(wr_ref[e]))
             + _dot3(h2, _split_bf16(wo_ref[e]))
             + b_ref[e:e + 1, :])
        y = jax.nn.relu(y)
        out_ref[e] = y
        osp_ref[e, 0] = y[:, :_HALF]
        osp_ref[e, 1] = y[:, _HALF:]


def _layer0_call(aggp, h, wr0, wo0, b0):
    grid = N_NODES // _BLK
    return pl.pallas_call(
        _layer0_kernel,
        grid=(grid,),
        in_specs=[
            pl.BlockSpec((_NC, _BLK, _HALF), lambda i: (0, i, 0)),
            pl.BlockSpec((_BLK, HID), lambda i: (i, 0)),
            pl.BlockSpec((N_EXPERTS, HID, HID), lambda i: (0, 0, 0)),
            pl.BlockSpec((N_EXPERTS, HID, HID), lambda i: (0, 0, 0)),
            pl.BlockSpec((N_EXPERTS, HID), lambda i: (0, 0)),
        ],
        out_specs=[
            pl.BlockSpec((N_EXPERTS, _BLK, HID), lambda i: (0, i, 0)),
            pl.BlockSpec((N_EXPERTS, _NC, _BLK, _HALF), lambda i: (0, 0, i, 0)),
        ],
        out_shape=[
            jax.ShapeDtypeStruct((N_EXPERTS, N_NODES, HID), _F32),
            jax.ShapeDtypeStruct((N_EXPERTS, _NC, N_NODES, _HALF), _F32),
        ],
    )(aggp, h, wr0, wo0, b0)


def _layer1_kernel(*refs):
    aggp_refs = refs[:N_EXPERTS]
    y0_ref, wr_ref, wo_ref, b_ref, out_ref, osp_ref = refs[N_EXPERTS:]
    for e in range(N_EXPERTS):
        agg2 = _split_bf16(_cat_agg(aggp_refs[e]))
        y = (_dot3(agg2, _split_bf16(wr_ref[e]))
             + _dot3(_split_bf16(y0_ref[e]), _split_bf16(wo_ref[e]))
             + b_ref[e:e + 1, :])
        y = jax.nn.relu(y)
        out_ref[e] = y
        osp_ref[0, e] = y[:, :_HALF]
        osp_ref[1, e] = y[:, _HALF:]


def _layer1_call(aggp_list, y0, wr1, wo1, b1):
    grid = N_NODES // _BLK
    in_specs = (
        [pl.BlockSpec((_NC, _BLK, _HALF), lambda i: (0, i, 0))] * N_EXPERTS
        + [
            pl.BlockSpec((N_EXPERTS, _BLK, HID), lambda i: (0, i, 0)),
            pl.BlockSpec((N_EXPERTS, HID, HID), lambda i: (0, 0, 0)),
            pl.BlockSpec((N_EXPERTS, HID, HID), lambda i: (0, 0, 0)),
            pl.BlockSpec((N_EXPERTS, HID), lambda i: (0, 0)),
        ]
    )
    return pl.pallas_call(
        _layer1_kernel,
        grid=(grid,),
        in_specs=in_specs,
        out_specs=[
            pl.BlockSpec((N_EXPERTS, _BLK, HID), lambda i: (0, i, 0)),
            pl.BlockSpec((_NC, N_EXPERTS, _BLK, _HALF), lambda i: (0, 0, i, 0)),
        ],
        out_shape=[
            jax.ShapeDtypeStruct((N_EXPERTS, N_NODES, HID), _F32),
            jax.ShapeDtypeStruct((_NC, N_EXPERTS, N_NODES, _HALF), _F32),
        ],
    )(*aggp_list, y0, wr1, wo1, b1)


def _final_kernel(a0p_ref, a1p_ref, y1_ref, w_ref, ti_ref, wr2_ref, wo2_ref,
                  b2_ref, out_ref):
    a02 = _split_bf16(_cat_agg(a0p_ref))
    a12 = _split_bf16(_cat_agg(a1p_ref))
    w = w_ref[...]
    ti = ti_ref[...]
    acc = jnp.zeros((w.shape[0], OUT), _F32)
    for e in range(N_EXPERTS):
        wr2e = _split_bf16(wr2_ref[e])
        r0 = _dot3(a02, wr2e)
        r1 = _dot3(a12, wr2e)
        se = _dot3(_split_bf16(y1_ref[e]), _split_bf16(wo2_ref[e]))
        base = se + b2_ref[e:e + 1, :]
        sel0 = (ti[:, 0:1] == e).astype(_F32)
        sel1 = (ti[:, 1:2] == e).astype(_F32)
        acc = acc + w[:, 0:1] * sel0 * (r0 + base) + w[:, 1:2] * sel1 * (r1 + base)
    out_ref[...] = acc


def _final_call(a0p, a1p, y1, wts, topi, wr2, wo2, b2):
    grid = N_NODES // _BLK
    return pl.pallas_call(
        _final_kernel,
        grid=(grid,),
        in_specs=[
            pl.BlockSpec((_NC, _BLK, _HALF), lambda i: (0, i, 0)),
            pl.BlockSpec((_NC, _BLK, _HALF), lambda i: (0, i, 0)),
            pl.BlockSpec((N_EXPERTS, _BLK, HID), lambda i: (0, i, 0)),
            pl.BlockSpec((_BLK, 2), lambda i: (i, 0)),
            pl.BlockSpec((_BLK, 2), lambda i: (i, 0)),
            pl.BlockSpec((N_EXPERTS, OUT, HID), lambda i: (0, 0, 0)),
            pl.BlockSpec((N_EXPERTS, OUT, HID), lambda i: (0, 0, 0)),
            pl.BlockSpec((N_EXPERTS, OUT), lambda i: (0, 0)),
        ],
        out_specs=pl.BlockSpec((_BLK, OUT), lambda i: (i, 0)),
        out_shape=jax.ShapeDtypeStruct((N_NODES, OUT), _F32),
    )(a0p, a1p, y1, wts, topi, wr2, wo2, b2)


def kernel(x, edge_index, batch, enc_w1, enc_b1, enc_w2, enc_b2,
           rout_w1, rout_b1, ln_g, ln_b, rout_w2, rout_b2, centers,
           ewr0, ewo0, eb0, ewr1, ewo1, eb1, ewr2, ewo2, eb2):
    src = edge_index[0]
    dst = edge_index[1]
    srcf = jnp.pad(src, (0, _E_PAD - N_EDGES))
    dstf = jnp.pad(dst, (0, _E_PAD - N_EDGES), constant_values=_DUMP)
    dst2d = dstf.reshape(_E_PAD // _CHUNK, _CHUNK)
    src2d = srcf.reshape(_E_PAD // _CHUNK, _CHUNK)
    zrows = jnp.zeros((_ROWS_PER_TILE, _HALF), _F32)
    batch1 = batch[:, None]
    src2d_real = src.reshape(N_EDGES // _CHUNK, _CHUNK)

    stats = _stats_call(batch1, src2d_real)
    h, hsplit, wts, topi = _router_call(
        x, batch1, stats, enc_w1, enc_b1[None, :], enc_w2, enc_b2[None, :],
        rout_w1[:, :HID], rout_w1[:, HID:], rout_b1[None, :],
        ln_g[None, :], ln_b[None, :], rout_w2, rout_b2[None, :],
        centers[None, :])

    agg0 = _segsum_multi([hsplit.reshape(_NC * N_NODES, _HALF)],
                         src2d, dst2d, zrows)[0]
    y0, y0split = _layer0_call(agg0, h, ewr0, ewo0, eb0)

    agg1 = []
    for e0 in range(0, N_EXPERTS, 2):
        pair = _segsum_multi(
            [y0split[e0 + i].reshape(_NC * N_NODES, _HALF) for i in range(2)],
            src2d, dst2d, zrows)
        agg1 += [pair[i] for i in range(2)]
    y1, y1split = _layer1_call(agg1, y0, ewr1, ewo1, eb1)

    topi2 = jnp.pad(topi.T, ((0, 0), (0, _AGG_ROWS - N_NODES))).reshape(
        2 * (_AGG_ROWS // _CHUNK), _CHUNK)
    gix = _gidx(srcf, dstf, topi2)
    y1flat = y1split.reshape(_NC * N_EXPERTS * N_NODES, _HALF)
    akp = _segsum2k(y1flat,
                    gix.reshape(2, _NC, _E_PAD // _CHUNK, _CHUNK),
                    dst2d, zrows)

    return _final_call(akp[0], akp[1], y1, wts, topi, ewr2, ewo2, eb2)


# speedup vs baseline: 1.0856x; 1.0646x over previous
import jax
import jax.numpy as jnp
from jax import lax
from jax.experimental import pallas as pl
from jax.experimental.pallas import tpu as pltpu
from jax.experimental.pallas import tpu_sc as plsc

N_NODES = 10000
N_EDGES = 320000
N_GRAPHS = 16
N_EXPERTS = 8
HID = 128
OUT = 6

_NC = 2
_NS = 16

_HALF = HID // 2
_CHUNK = 128
_E_PER_T = 20480
_E_PAD = _NS * _E_PER_T
_NCH = _E_PER_T // _CHUNK
_NBUF = 6
_NBUF_S = 3
_QCH = 40

_AGG_ROWS = 10240
_ROWS_PER_TILE = _AGG_ROWS // _NS
_DUMP = _AGG_ROWS - 1

_F32 = jnp.float32
_HIGH = jax.lax.Precision.HIGHEST


def _sc_mesh():
    return plsc.VectorSubcoreMesh(core_axis_name="c", subcore_axis_name="s")


def _edge_pass(table, src_all, dst_all, rows, agg, gsems, ssems, nbuf, nch):
    g = [None] * nbuf
    sc = [None] * nbuf

    def start_gather(j):
        b = j % nbuf
        g[b] = pltpu.async_copy(table.at[src_all.at[j]], rows.at[b], gsems[b])

    def start_scatter(j):
        b = j % nbuf
        sc[b] = pltpu.async_copy(rows.at[b], agg.at[dst_all.at[j]], ssems[b],
                                 add=True)

    for j in range(nbuf - 1):
        start_gather(j)
    for j in range(nch):
        b = j % nbuf
        g[b].wait()
        start_scatter(j)
        jn = j + nbuf - 1
        if jn < nch:
            bn = jn % nbuf
            if sc[bn] is not None:
                sc[bn].wait()
            start_gather(jn)
    for j in range(nch - nbuf, nch):
        if sc[j % nbuf] is not None:
            sc[j % nbuf].wait()


def _writeback_and_zero(out_slot, zrows, rows, agg, c, s, zero_after, sem):
    h = pltpu.async_copy(
        agg.at[pl.ds(s * _ROWS_PER_TILE, _ROWS_PER_TILE)],
        out_slot.at[pl.ds(s * _ROWS_PER_TILE, _ROWS_PER_TILE)], sem)
    h.wait()
    if zero_after:
        pltpu.sync_copy(
            zrows, agg.at[pl.ds(s * _ROWS_PER_TILE, _ROWS_PER_TILE)])
        plsc.subcore_barrier()


def _make_multi_body(ntab):
    def body(*refs):
        tables = refs[:ntab]
        src2d, dstix, zrows, out = refs[ntab:ntab + 4]
        src_q, dst_q, rows, agg, tab_sh = refs[ntab + 4:ntab + 9]
        sems = refs[ntab + 9:]
        gsems = sems[:_NBUF_S]
        ssems = sems[_NBUF_S:]
        c = lax.axis_index("c")
        s = lax.axis_index("s")
        t_rows = N_NODES // _NS
        pltpu.sync_copy(zrows, agg.at[pl.ds(s * _ROWS_PER_TILE, _ROWS_PER_TILE)])
        for t in range(ntab):
            pltpu.sync_copy(
                tables[t].at[pl.ds(c * N_NODES + s * t_rows, t_rows)],
                tab_sh.at[pl.ds(s * t_rows, t_rows)])
            plsc.subcore_barrier()
            for q in range(_NCH // _QCH):
                base_row = s * _NCH + q * _QCH
                pltpu.sync_copy(src2d.at[pl.ds(base_row, _QCH)], src_q)
                pltpu.sync_copy(dstix.at[pl.ds(base_row, _QCH)], dst_q)
                _edge_pass(tab_sh, src_q, dst_q, rows, agg, gsems, ssems,
                           _NBUF_S, _QCH)
            plsc.subcore_barrier()
            _writeback_and_zero(out.at[t, c], zrows, rows, agg, c, s,
                                zero_after=(t + 1 < ntab), sem=sems[0])
    return body


def _segsum_multi(tables, src2d, dst2d, zrows):
    ntab = len(tables)
    f = pl.kernel(
        _make_multi_body(ntab),
        out_type=jax.ShapeDtypeStruct((ntab, _NC, _AGG_ROWS, _HALF), _F32),
        mesh=_sc_mesh(),
        scratch_types=(
            [
                pltpu.VMEM((_QCH, _CHUNK), jnp.int32),
                pltpu.VMEM((_QCH, _CHUNK), jnp.int32),
                pltpu.VMEM((_NBUF_S, _CHUNK, _HALF), _F32),
                pltpu.VMEM_SHARED((_AGG_ROWS, _HALF), _F32),
                pltpu.VMEM_SHARED((N_NODES, _HALF), _F32),
            ]
            + [pltpu.SemaphoreType.DMA] * (2 * _NBUF_S)
        ),
        compiler_params=pltpu.CompilerParams(use_tc_tiling_on_sc=False),
    )
    return f(*tables, src2d, dst2d, zrows)


def _segsum2k_body(table, srcix2k, dstix, zrows, out,
                   src_all, dst_all, rows, agg, *sems):
    gsems = sems[:_NBUF]
    ssems = sems[_NBUF:]
    c = lax.axis_index("c")
    s = lax.axis_index("s")
    base_row = s * _NCH
    pltpu.sync_copy(zrows, agg.at[pl.ds(s * _ROWS_PER_TILE, _ROWS_PER_TILE)])
    pltpu.sync_copy(dstix.at[pl.ds(base_row, _NCH)], dst_all)
    for k in range(2):
        pltpu.sync_copy(srcix2k.at[k, c, pl.ds(base_row, _NCH)], src_all)
        plsc.subcore_barrier()
        _edge_pass(table, src_all, dst_all, rows, agg, gsems, ssems,
                   _NBUF, _NCH)
        plsc.subcore_barrier()
        _writeback_and_zero(out.at[k, c], zrows, rows, agg, c, s,
                            zero_after=(k == 0), sem=sems[0])


def _segsum2k(table_flat, srcix2k, dst2d, zrows):
    f = pl.kernel(
        _segsum2k_body,
        out_type=jax.ShapeDtypeStruct((2, _NC, _AGG_ROWS, _HALF), _F32),
        mesh=_sc_mesh(),
        scratch_types=(
            [
                pltpu.VMEM((_NCH, _CHUNK), jnp.int32),
                pltpu.VMEM((_NCH, _CHUNK), jnp.int32),
                pltpu.VMEM((_NBUF, _CHUNK, _HALF), _F32),
                pltpu.VMEM_SHARED((_AGG_ROWS, _HALF), _F32),
            ]
            + [pltpu.SemaphoreType.DMA] * (2 * _NBUF)
        ),
        compiler_params=pltpu.CompilerParams(use_tc_tiling_on_sc=False),
    )
    return f(table_flat, srcix2k, dst2d, zrows)


def _gidx_body(srcf, dstf, topi2, out, srcb, dstb, topi_v, gbuf, sem):
    c = lax.axis_index("c")
    s = lax.axis_index("s")
    base = s * _E_PER_T
    pltpu.sync_copy(srcf.at[pl.ds(base, _E_PER_T)], srcb)
    pltpu.sync_copy(dstf.at[pl.ds(base, _E_PER_T)], dstb)
    pltpu.sync_copy(topi2, topi_v)
    coff = c * (N_EXPERTS * N_NODES)
    nrow = _AGG_ROWS // _CHUNK
    for k in range(2):
        def step(i, carry):
            sv = srcb[pl.ds(i * 16, 16)]
            dv = dstb[pl.ds(i * 16, 16)]
            ev = plsc.load_gather(
                topi_v, [k * nrow + lax.shift_right_logical(dv, 7),
                         lax.bitwise_and(dv, 127)])
            gbuf[pl.ds(i * 16, 16)] = ev * N_NODES + sv + coff
            return carry
        lax.fori_loop(0, _E_PER_T // 16, step, 0)
        pltpu.sync_copy(gbuf, out.at[k, c, pl.ds(base, _E_PER_T)])


def _gidx(srcf, dstf, topi2):
    f = pl.kernel(
        _gidx_body,
        out_type=jax.ShapeDtypeStruct((2, _NC, _E_PAD), jnp.int32),
        mesh=_sc_mesh(),
        scratch_types=[
            pltpu.VMEM((_E_PER_T,), jnp.int32),
            pltpu.VMEM((_E_PER_T,), jnp.int32),
            pltpu.VMEM((2 * (_AGG_ROWS // _CHUNK), _CHUNK), jnp.int32),
            pltpu.VMEM((_E_PER_T,), jnp.int32),
            pltpu.SemaphoreType.DMA,
        ],
        compiler_params=pltpu.CompilerParams(needs_layout_passes=False),
    )
    return f(srcf, dstf, topi2)


_BLK = 1000


def _stats_kernel(batch_ref, src_ref, stats_ref):
    b = batch_ref[...]
    srows = src_ref[...].astype(_F32)
    gids = lax.broadcasted_iota(jnp.int32, (1, N_GRAPHS), 1)
    oh = (b == gids).astype(_F32)
    n_per = jnp.sum(oh, axis=0, keepdims=True)
    ii = lax.broadcasted_iota(jnp.int32, (N_GRAPHS, N_GRAPHS), 0)
    jj = lax.broadcasted_iota(jnp.int32, (N_GRAPHS, N_GRAPHS), 1)
    tri = (ii <= jj).astype(_F32)
    bounds = lax.dot_general(n_per, tri, (((1,), (0,)), ((), ())),
                             preferred_element_type=_F32, precision=_HIGH)
    cnt_prev = jnp.zeros((1, 1), _F32)
    e_list = []
    for g in range(N_GRAPHS):
        cnt = jnp.sum(jnp.where(srows < bounds[:, g:g + 1], 1.0, 0.0)).reshape(1, 1)
        e_list.append(cnt - cnt_prev)
        cnt_prev = cnt
    e_per = jnp.concatenate(e_list, axis=1)
    n = jnp.maximum(n_per, 1.0)
    e = jnp.maximum(e_per, 0.0)
    log_n = jnp.log(n)
    log_e = jnp.log1p(e)
    log_n_norm = ((log_n - jnp.min(log_n))
                  / (jnp.max(log_n) - jnp.min(log_n) + 1e-6))
    def _std(v):
        m = jnp.mean(v)
        sd = jnp.sqrt(jnp.mean((v - m) ** 2))
        return (v - m) / (sd + 1e-6)
    stats_ref[...] = jnp.concatenate(
        [_std(log_n), _std(log_e), log_n_norm], axis=0)


def _stats_call(batch1, src2d_real):
    return pl.pallas_call(
        _stats_kernel,
        out_shape=jax.ShapeDtypeStruct((3, N_GRAPHS), _F32),
    )(batch1, src2d_real)


def _router_kernel(x_ref, b_ref, stats_ref, ew1_ref, eb1_ref, ew2_ref, eb2_ref,
                   rw1h_ref, rw1f_ref, rb1_ref, lg_ref, lb_ref, rw2_ref, rb2_ref,
                   cen_ref, h_ref, hs_ref, w_ref, ti_ref):
    xs = x_ref[...][:, 4:10]
    h1 = jax.nn.relu(
        lax.dot_general(xs, ew1_ref[...], (((1,), (1,)), ((), ())),
                        preferred_element_type=_F32, precision=_HIGH)
        + eb1_ref[...])
    h = lax.dot_general(h1, ew2_ref[...], (((1,), (1,)), ((), ())),
                        preferred_element_type=_F32, precision=_HIGH) + eb2_ref[...]
    h_ref[...] = h
    hs_ref[0] = h[:, :_HALF]
    hs_ref[1] = h[:, _HALF:]
    gids = lax.broadcasted_iota(jnp.int32, (1, N_GRAPHS), 1)
    oh = (b_ref[...] == gids).astype(_F32)
    nf = lax.dot_general(oh, stats_ref[...], (((1,), (1,)), ((), ())),
                         preferred_element_type=_F32, precision=_HIGH)
    r = (lax.dot_general(h, rw1h_ref[...], (((1,), (1,)), ((), ())),
                         preferred_element_type=_F32, precision=_HIGH)
         + lax.dot_general(nf[:, 0:2], rw1f_ref[...], (((1,), (1,)), ((), ())),
                           preferred_element_type=_F32, precision=_HIGH)
         + rb1_ref[...])
    mu = jnp.mean(r, axis=-1, keepdims=True)
    var = jnp.mean((r - mu) ** 2, axis=-1, keepdims=True)
    r = (r - mu) * lax.rsqrt(var + 1e-5) * lg_ref[...] + lb_ref[...]
    r = jax.nn.relu(r)
    learned = lax.dot_general(r, rw2_ref[...], (((1,), (1,)), ((), ())),
                              preferred_element_type=_F32, precision=_HIGH) + rb2_ref[...]
    prior = -((nf[:, 2:3] - cen_ref[...]) ** 2)
    logits = 0.65 * learned + 0.35 * prior
    m = jnp.max(logits, axis=-1, keepdims=True)
    ex = jnp.exp(logits - m)
    p = ex / jnp.sum(ex, axis=-1, keepdims=True)
    ids = lax.broadcasted_iota(jnp.int32, p.shape, 1)
    m1 = jnp.max(p, axis=-1, keepdims=True)
    i1 = jnp.min(jnp.where(p == m1, ids, N_EXPERTS), axis=-1, keepdims=True)
    p2 = jnp.where(ids == i1, -1.0, p)
    m2 = jnp.max(p2, axis=-1, keepdims=True)
    i2 = jnp.min(jnp.where(p2 == m2, ids, N_EXPERTS), axis=-1, keepdims=True)
    tot = m1 + m2
    w_ref[...] = jnp.concatenate([m1 / tot, m2 / tot], axis=-1)
    ti_ref[...] = jnp.concatenate([i1, i2], axis=-1)


def _router_call(x, batch1, stats, enc_w1, enc_b1, enc_w2, enc_b2,
                 rw1h, rw1f, rb1, ln_g, ln_b, rw2, rb2, cen):
    grid = N_NODES // _BLK
    bs_in = [
        pl.BlockSpec((_BLK, 16), lambda i: (i, 0)),
        pl.BlockSpec((_BLK, 1), lambda i: (i, 0)),
        pl.BlockSpec((3, N_GRAPHS), lambda i: (0, 0)),
        pl.BlockSpec((HID, 6), lambda i: (0, 0)),
        pl.BlockSpec((1, HID), lambda i: (0, 0)),
        pl.BlockSpec((HID, HID), lambda i: (0, 0)),
        pl.BlockSpec((1, HID), lambda i: (0, 0)),
        pl.BlockSpec((HID, HID), lambda i: (0, 0)),
        pl.BlockSpec((HID, 2), lambda i: (0, 0)),
        pl.BlockSpec((1, HID), lambda i: (0, 0)),
        pl.BlockSpec((1, HID), lambda i: (0, 0)),
        pl.BlockSpec((1, HID), lambda i: (0, 0)),
        pl.BlockSpec((N_EXPERTS, HID), lambda i: (0, 0)),
        pl.BlockSpec((1, N_EXPERTS), lambda i: (0, 0)),
        pl.BlockSpec((1, N_EXPERTS), lambda i: (0, 0)),
    ]
    bs_out = [
        pl.BlockSpec((_BLK, HID), lambda i: (i, 0)),
        pl.BlockSpec((_NC, _BLK, _HALF), lambda i: (0, i, 0)),
        pl.BlockSpec((_BLK, 2), lambda i: (i, 0)),
        pl.BlockSpec((_BLK, 2), lambda i: (i, 0)),
    ]
    return pl.pallas_call(
        _router_kernel,
        grid=(grid,),
        in_specs=bs_in,
        out_specs=bs_out,
        out_shape=[
            jax.ShapeDtypeStruct((N_NODES, HID), _F32),
            jax.ShapeDtypeStruct((_NC, N_NODES, _HALF), _F32),
            jax.ShapeDtypeStruct((N_NODES, 2), _F32),
            jax.ShapeDtypeStruct((N_NODES, 2), jnp.int32),
        ],
    )(x, batch1, stats, enc_w1, enc_b1, enc_w2, enc_b2,
      rw1h, rw1f, rb1, ln_g, ln_b, rw2, rb2, cen)


def _cat_agg(aggp_ref):
    return jnp.concatenate([aggp_ref[0], aggp_ref[1]], axis=-1)


def _split_bf16(a):
    return a


def _dot3(a, b):
    dn = (((1,), (1,)), ((), ()))
    return lax.dot_general(a, b, dn, preferred_element_type=_F32,
                           precision=_HIGH)


def _layer0_kernel(aggp_ref, h_ref, wr_ref, wo_ref, b_ref, out_ref, osp_ref):
    agg2 = _split_bf16(_cat_agg(aggp_ref))
    h2 = _split_bf16(h_ref[...])
    for e in range(N_EXPERTS):
        y = (_dot3(agg2, _split_bf16(wr_ref[e]))
             + _dot3(h2, _split_bf16(wo_ref[e]))
             + b_ref[e:e + 1, :])
        y = jax.nn.relu(y)
        out_ref[e] = y
        osp_ref[e, 0] = y[:, :_HALF]
        osp_ref[e, 1] = y[:, _HALF:]


def _layer0_call(aggp, h, wr0, wo0, b0):
    grid = N_NODES // _BLK
    return pl.pallas_call(
        _layer0_kernel,
        grid=(grid,),
        in_specs=[
            pl.BlockSpec((_NC, _BLK, _HALF), lambda i: (0, i, 0)),
            pl.BlockSpec((_BLK, HID), lambda i: (i, 0)),
            pl.BlockSpec((N_EXPERTS, HID, HID), lambda i: (0, 0, 0)),
            pl.BlockSpec((N_EXPERTS, HID, HID), lambda i: (0, 0, 0)),
            pl.BlockSpec((N_EXPERTS, HID), lambda i: (0, 0)),
        ],
        out_specs=[
            pl.BlockSpec((N_EXPERTS, _BLK, HID), lambda i: (0, i, 0)),
            pl.BlockSpec((N_EXPERTS, _NC, _BLK, _HALF), lambda i: (0, 0, i, 0)),
        ],
        out_shape=[
            jax.ShapeDtypeStruct((N_EXPERTS, N_NODES, HID), _F32),
            jax.ShapeDtypeStruct((N_EXPERTS, _NC, N_NODES, _HALF), _F32),
        ],
    )(aggp, h, wr0, wo0, b0)


def _layer1_kernel(*refs):
    aggp_refs = refs[:N_EXPERTS]
    y0_ref, wr_ref, wo_ref, b_ref, out_ref, osp_ref = refs[N_EXPERTS:]
    for e in range(N_EXPERTS):
        agg2 = _split_bf16(_cat_agg(aggp_refs[e]))
        y = (_dot3(agg2, _split_bf16(wr_ref[e]))
             + _dot3(_split_bf16(y0_ref[e]), _split_bf16(wo_ref[e]))
             + b_ref[e:e + 1, :])
        y = jax.nn.relu(y)
        out_ref[e] = y
        osp_ref[0, e] = y[:, :_HALF]
        osp_ref[1, e] = y[:, _HALF:]


def _layer1_call(aggp_list, y0, wr1, wo1, b1):
    grid = N_NODES // _BLK
    in_specs = (
        [pl.BlockSpec((_NC, _BLK, _HALF), lambda i: (0, i, 0))] * N_EXPERTS
        + [
            pl.BlockSpec((N_EXPERTS, _BLK, HID), lambda i: (0, i, 0)),
            pl.BlockSpec((N_EXPERTS, HID, HID), lambda i: (0, 0, 0)),
            pl.BlockSpec((N_EXPERTS, HID, HID), lambda i: (0, 0, 0)),
            pl.BlockSpec((N_EXPERTS, HID), lambda i: (0, 0)),
        ]
    )
    return pl.pallas_call(
        _layer1_kernel,
        grid=(grid,),
        in_specs=in_specs,
        out_specs=[
            pl.BlockSpec((N_EXPERTS, _BLK, HID), lambda i: (0, i, 0)),
            pl.BlockSpec((_NC, N_EXPERTS, _BLK, _HALF), lambda i: (0, 0, i, 0)),
        ],
        out_shape=[
            jax.ShapeDtypeStruct((N_EXPERTS, N_NODES, HID), _F32),
            jax.ShapeDtypeStruct((_NC, N_EXPERTS, N_NODES, _HALF), _F32),
        ],
    )(*aggp_list, y0, wr1, wo1, b1)


def _dot_d(a, b):
    return lax.dot_general(a, b, (((1,), (1,)), ((), ())),
                           preferred_element_type=_F32)


def _final_kernel(a0p_ref, a1p_ref, y1_ref, w_ref, ti_ref, wr2_ref, wo2_ref,
                  b2_ref, out_ref):
    a02 = _split_bf16(_cat_agg(a0p_ref))
    a12 = _split_bf16(_cat_agg(a1p_ref))
    w = w_ref[...]
    ti = ti_ref[...]
    acc = jnp.zeros((w.shape[0], OUT), _F32)
    for e in range(N_EXPERTS):
        wr2e = _split_bf16(wr2_ref[e])
        r0 = _dot_d(a02, wr2e)
        r1 = _dot_d(a12, wr2e)
        se = _dot_d(_split_bf16(y1_ref[e]), _split_bf16(wo2_ref[e]))
        base = se + b2_ref[e:e + 1, :]
        sel0 = (ti[:, 0:1] == e).astype(_F32)
        sel1 = (ti[:, 1:2] == e).astype(_F32)
        acc = acc + w[:, 0:1] * sel0 * (r0 + base) + w[:, 1:2] * sel1 * (r1 + base)
    out_ref[...] = acc


def _final_call(a0p, a1p, y1, wts, topi, wr2, wo2, b2):
    grid = N_NODES // _BLK
    return pl.pallas_call(
        _final_kernel,
        grid=(grid,),
        in_specs=[
            pl.BlockSpec((_NC, _BLK, _HALF), lambda i: (0, i, 0)),
            pl.BlockSpec((_NC, _BLK, _HALF), lambda i: (0, i, 0)),
            pl.BlockSpec((N_EXPERTS, _BLK, HID), lambda i: (0, i, 0)),
            pl.BlockSpec((_BLK, 2), lambda i: (i, 0)),
            pl.BlockSpec((_BLK, 2), lambda i: (i, 0)),
            pl.BlockSpec((N_EXPERTS, OUT, HID), lambda i: (0, 0, 0)),
            pl.BlockSpec((N_EXPERTS, OUT, HID), lambda i: (0, 0, 0)),
            pl.BlockSpec((N_EXPERTS, OUT), lambda i: (0, 0)),
        ],
        out_specs=pl.BlockSpec((_BLK, OUT), lambda i: (i, 0)),
        out_shape=jax.ShapeDtypeStruct((N_NODES, OUT), _F32),
    )(a0p, a1p, y1, wts, topi, wr2, wo2, b2)


def kernel(x, edge_index, batch, enc_w1, enc_b1, enc_w2, enc_b2,
           rout_w1, rout_b1, ln_g, ln_b, rout_w2, rout_b2, centers,
           ewr0, ewo0, eb0, ewr1, ewo1, eb1, ewr2, ewo2, eb2):
    src = edge_index[0]
    dst = edge_index[1]
    srcf = jnp.pad(src, (0, _E_PAD - N_EDGES))
    dstf = jnp.pad(dst, (0, _E_PAD - N_EDGES), constant_values=_DUMP)
    dst2d = dstf.reshape(_E_PAD // _CHUNK, _CHUNK)
    src2d = srcf.reshape(_E_PAD // _CHUNK, _CHUNK)
    zrows = jnp.zeros((_ROWS_PER_TILE, _HALF), _F32)
    batch1 = batch[:, None]
    src2d_real = src.reshape(N_EDGES // _CHUNK, _CHUNK)

    stats = _stats_call(batch1, src2d_real)
    h, hsplit, wts, topi = _router_call(
        x, batch1, stats, enc_w1, enc_b1[None, :], enc_w2, enc_b2[None, :],
        rout_w1[:, :HID], rout_w1[:, HID:], rout_b1[None, :],
        ln_g[None, :], ln_b[None, :], rout_w2, rout_b2[None, :],
        centers[None, :])

    agg0 = _segsum_multi([hsplit.reshape(_NC * N_NODES, _HALF)],
                         src2d, dst2d, zrows)[0]
    y0, y0split = _layer0_call(agg0, h, ewr0, ewo0, eb0)

    agg1 = []
    for e0 in range(0, N_EXPERTS, 2):
        pair = _segsum_multi(
            [y0split[e0 + i].reshape(_NC * N_NODES, _HALF) for i in range(2)],
            src2d, dst2d, zrows)
        agg1 += [pair[i] for i in range(2)]
    y1, y1split = _layer1_call(agg1, y0, ewr1, ewo1, eb1)

    topi2 = jnp.pad(topi.T, ((0, 0), (0, _AGG_ROWS - N_NODES))).reshape(
        2 * (_AGG_ROWS // _CHUNK), _CHUNK)
    gix = _gidx(srcf, dstf, topi2)
    y1flat = y1split.reshape(_NC * N_EXPERTS * N_NODES, _HALF)
    akp = _segsum2k(y1flat,
                    gix.reshape(2, _NC, _E_PAD // _CHUNK, _CHUNK),
                    dst2d, zrows)

    return _final_call(akp[0], akp[1], y1, wts, topi, ewr2, ewo2, eb2)
